# Initial kernel scaffold; baseline (speedup 1.0000x reference)
import jax
import jax.numpy as jnp
from jax.experimental import pallas as pl


def kernel(X):
    raise NotImplementedError("write your pallas kernel here")


# speedup vs baseline: 2.1048x; 2.1048x over previous
import functools

import jax
import jax.numpy as jnp
from jax.experimental import pallas as pl
from jax.experimental.pallas import tpu as pltpu

NS_ITERS = 5
EPS = 1e-05
NCORES = 2


def _stats_kernel(x_ref, gram_ref, sum_ref):
    j = pl.program_id(1)

    @pl.when(j == 0)
    def _init():
        gram_ref[0] = jnp.zeros_like(gram_ref[0])
        sum_ref[0] = jnp.zeros_like(sum_ref[0])

    x = x_ref[0]
    gram_ref[0] += jax.lax.dot_general(
        x, x, (((1,), (1,)), ((), ())), preferred_element_type=jnp.float32
    )
    sum_ref[0] += jnp.sum(x, axis=1, keepdims=True)


def _apply_kernel(m_total, gram_ref, sum_ref, x_ref, o_ref, wm_ref, wb_ref):
    j = pl.program_id(1)

    @pl.when(j == 0)
    def _compute_wm():
        d = gram_ref.shape[1]
        gram = gram_ref[0] + gram_ref[1]
        s = sum_ref[0] + sum_ref[1]
        inv_m = 1.0 / jnp.float32(m_total)
        mean = s * inv_m
        rows = jax.lax.broadcasted_iota(jnp.int32, (d, d), 0)
        cols = jax.lax.broadcasted_iota(jnp.int32, (d, d), 1)
        eye = jnp.where(rows == cols, jnp.float32(1.0), jnp.float32(0.0))
        outer = jax.lax.dot_general(
            mean, mean, (((1,), (1,)), ((), ())),
            preferred_element_type=jnp.float32,
        )
        sigma = gram * inv_m - outer + EPS * eye
        tr = jnp.sum(jnp.where(rows == cols, sigma, jnp.float32(0.0)))
        r_tr = 1.0 / tr
        sigma_n = sigma * r_tr
        p = eye
        for _ in range(NS_ITERS):
            p2 = jnp.dot(p, p, preferred_element_type=jnp.float32)
            p3 = jnp.dot(p2, p, preferred_element_type=jnp.float32)
            p = 1.5 * p - 0.5 * jnp.dot(
                p3, sigma_n, preferred_element_type=jnp.float32
            )
        wm = p * jnp.sqrt(r_tr)
        wm_ref[...] = wm
        wb_ref[...] = jnp.dot(wm, mean, preferred_element_type=jnp.float32)

    x = x_ref[0]
    o_ref[0] = (
        jnp.dot(wm_ref[...], x, preferred_element_type=jnp.float32)
        - wb_ref[...]
    )


def kernel(X):
    B, C, L = X.shape
    m_total = B * L
    blocks_per_core = B // NCORES

    grid = (NCORES, blocks_per_core)
    x_spec = pl.BlockSpec(
        (1, C, L), lambda i, j, nb=blocks_per_core: (i * nb + j, 0, 0)
    )

    gram_p, sum_p = pl.pallas_call(
        _stats_kernel,
        grid=grid,
        in_specs=[x_spec],
        out_specs=[
            pl.BlockSpec((1, C, C), lambda i, j: (i, 0, 0)),
            pl.BlockSpec((1, C, 1), lambda i, j: (i, 0, 0)),
        ],
        out_shape=[
            jax.ShapeDtypeStruct((NCORES, C, C), jnp.float32),
            jax.ShapeDtypeStruct((NCORES, C, 1), jnp.float32),
        ],
        compiler_params=pltpu.CompilerParams(
            dimension_semantics=("parallel", "arbitrary"),
        ),
        name="iternorm_stats",
    )(X)

    out = pl.pallas_call(
        functools.partial(_apply_kernel, m_total),
        grid=grid,
        in_specs=[
            pl.BlockSpec((NCORES, C, C), lambda i, j: (0, 0, 0)),
            pl.BlockSpec((NCORES, C, 1), lambda i, j: (0, 0, 0)),
            x_spec,
        ],
        out_specs=pl.BlockSpec(
            (1, C, L), lambda i, j, nb=blocks_per_core: (i * nb + j, 0, 0)
        ),
        out_shape=jax.ShapeDtypeStruct((B, C, L), jnp.float32),
        scratch_shapes=[
            pltpu.VMEM((C, C), jnp.float32),
            pltpu.VMEM((C, 1), jnp.float32),
        ],
        compiler_params=pltpu.CompilerParams(
            dimension_semantics=("parallel", "arbitrary"),
        ),
        name="iternorm_apply",
    )(gram_p, sum_p, X)

    return out


# speedup vs baseline: 2.5575x; 1.2151x over previous
import functools

import jax
import jax.numpy as jnp
from jax.experimental import pallas as pl
from jax.experimental.pallas import tpu as pltpu

NS_ITERS = 5
EPS = 1e-05
NCORES = 2


def _stats_kernel(x_ref, gram_ref, sum_ref):
    j = pl.program_id(1)

    @pl.when(j == 0)
    def _init():
        gram_ref[0] = jnp.zeros_like(gram_ref[0])
        sum_ref[0] = jnp.zeros_like(sum_ref[0])

    gram = gram_ref[0]
    ssum = sum_ref[0]
    for r in range(x_ref.shape[0]):
        x = x_ref[r]
        gram += jax.lax.dot_general(
            x, x, (((1,), (1,)), ((), ())), preferred_element_type=jnp.float32
        )
        ssum += jnp.sum(x, axis=1, keepdims=True)
    gram_ref[0] = gram
    sum_ref[0] = ssum


def _apply_kernel(m_total, gram_ref, sum_ref, x_ref, o_ref, wm_ref, wb_ref):
    j = pl.program_id(1)

    @pl.when(j == 0)
    def _compute_wm():
        d = gram_ref.shape[1]
        gram = gram_ref[0] + gram_ref[1]
        s = sum_ref[0] + sum_ref[1]
        inv_m = 1.0 / jnp.float32(m_total)
        mean = s * inv_m
        rows = jax.lax.broadcasted_iota(jnp.int32, (d, d), 0)
        cols = jax.lax.broadcasted_iota(jnp.int32, (d, d), 1)
        eye = jnp.where(rows == cols, jnp.float32(1.0), jnp.float32(0.0))
        outer = jax.lax.dot_general(
            mean, mean, (((1,), (1,)), ((), ())),
            preferred_element_type=jnp.float32,
        )
        sigma = gram * inv_m - outer + EPS * eye
        tr = jnp.sum(jnp.where(rows == cols, sigma, jnp.float32(0.0)))
        r_tr = 1.0 / tr
        sigma_n = sigma * r_tr
        p = eye
        for _ in range(NS_ITERS):
            p2 = jnp.dot(p, p, preferred_element_type=jnp.float32)
            ps = jnp.dot(p, sigma_n, preferred_element_type=jnp.float32)
            p = 1.5 * p - 0.5 * jnp.dot(
                p2, ps, preferred_element_type=jnp.float32
            )
        wm = p * jnp.sqrt(r_tr)
        wm_ref[...] = wm
        wb_ref[...] = jnp.dot(wm, mean, preferred_element_type=jnp.float32)

    for r in range(x_ref.shape[0]):
        o_ref[r] = (
            jnp.dot(wm_ref[...], x_ref[r], preferred_element_type=jnp.float32)
            - wb_ref[...]
        )


def kernel(X):
    B, C, L = X.shape
    m_total = B * L
    bb = 2
    blocks_per_core = B // (NCORES * bb)

    grid = (NCORES, blocks_per_core)
    x_spec = pl.BlockSpec(
        (bb, C, L), lambda i, j, nb=blocks_per_core: (i * nb + j, 0, 0)
    )

    gram_p, sum_p = pl.pallas_call(
        _stats_kernel,
        grid=grid,
        in_specs=[x_spec],
        out_specs=[
            pl.BlockSpec((1, C, C), lambda i, j: (i, 0, 0)),
            pl.BlockSpec((1, C, 1), lambda i, j: (i, 0, 0)),
        ],
        out_shape=[
            jax.ShapeDtypeStruct((NCORES, C, C), jnp.float32),
            jax.ShapeDtypeStruct((NCORES, C, 1), jnp.float32),
        ],
        compiler_params=pltpu.CompilerParams(
            dimension_semantics=("parallel", "arbitrary"),
        ),
        name="iternorm_stats",
    )(X)

    out = pl.pallas_call(
        functools.partial(_apply_kernel, m_total),
        grid=grid,
        in_specs=[
            pl.BlockSpec((NCORES, C, C), lambda i, j: (0, 0, 0)),
            pl.BlockSpec((NCORES, C, 1), lambda i, j: (0, 0, 0)),
            x_spec,
        ],
        out_specs=x_spec,
        out_shape=jax.ShapeDtypeStruct((B, C, L), jnp.float32),
        scratch_shapes=[
            pltpu.VMEM((C, C), jnp.float32),
            pltpu.VMEM((C, 1), jnp.float32),
        ],
        compiler_params=pltpu.CompilerParams(
            dimension_semantics=("parallel", "arbitrary"),
        ),
        name="iternorm_apply",
    )(gram_p, sum_p, X)

    return out


# speedup vs baseline: 2.7815x; 1.0876x over previous
import functools

import jax
import jax.numpy as jnp
from jax.experimental import pallas as pl
from jax.experimental.pallas import tpu as pltpu

NS_ITERS = 5
EPS = 1e-05
NCORES = 2


def _stats_kernel(x_ref, gram_ref, sum_ref):
    j = pl.program_id(1)

    @pl.when(j == 0)
    def _init():
        gram_ref[0] = jnp.zeros_like(gram_ref[0])
        sum_ref[0] = jnp.zeros_like(sum_ref[0])

    gram = gram_ref[0]
    ssum = sum_ref[0]
    for r in range(x_ref.shape[0]):
        x = x_ref[r]
        gram += jax.lax.dot_general(
            x, x, (((1,), (1,)), ((), ())), preferred_element_type=jnp.float32
        )
        ssum += jnp.sum(x, axis=1, keepdims=True)
    gram_ref[0] = gram
    sum_ref[0] = ssum


def _apply_kernel(m_total, gram_ref, sum_ref, x_ref, o_ref, wm_ref, wb_ref):
    j = pl.program_id(1)

    @pl.when(j == 0)
    def _compute_wm():
        d = gram_ref.shape[1]
        gram = gram_ref[0] + gram_ref[1]
        s = sum_ref[0] + sum_ref[1]
        inv_m = 1.0 / jnp.float32(m_total)
        mean = s * inv_m
        rows = jax.lax.broadcasted_iota(jnp.int32, (d, d), 0)
        cols = jax.lax.broadcasted_iota(jnp.int32, (d, d), 1)
        eye = jnp.where(rows == cols, jnp.float32(1.0), jnp.float32(0.0))
        outer = jax.lax.dot_general(
            mean, mean, (((1,), (1,)), ((), ())),
            preferred_element_type=jnp.float32,
        )
        sigma = gram * inv_m - outer + EPS * eye
        tr = jnp.sum(jnp.where(rows == cols, sigma, jnp.float32(0.0)))
        r_tr = 1.0 / tr
        sigma_n = sigma * r_tr
        p = eye
        for _ in range(NS_ITERS):
            p2 = jnp.dot(p, p, preferred_element_type=jnp.float32)
            ps = jnp.dot(p, sigma_n, preferred_element_type=jnp.float32)
            p = 1.5 * p - 0.5 * jnp.dot(
                p2, ps, preferred_element_type=jnp.float32
            )
        wm = p * jnp.sqrt(r_tr)
        wm_ref[...] = wm
        wb_ref[...] = jnp.dot(wm, mean, preferred_element_type=jnp.float32)

    for r in range(x_ref.shape[0]):
        o_ref[r] = (
            jnp.dot(wm_ref[...], x_ref[r], preferred_element_type=jnp.float32)
            - wb_ref[...]
        )


def kernel(X):
    B, C, L = X.shape
    m_total = B * L
    bb = 4
    blocks_per_core = B // (NCORES * bb)

    grid = (NCORES, blocks_per_core)
    x_spec = pl.BlockSpec(
        (bb, C, L), lambda i, j, nb=blocks_per_core: (i * nb + j, 0, 0)
    )

    gram_p, sum_p = pl.pallas_call(
        _stats_kernel,
        grid=grid,
        in_specs=[x_spec],
        out_specs=[
            pl.BlockSpec((1, C, C), lambda i, j: (i, 0, 0)),
            pl.BlockSpec((1, C, 1), lambda i, j: (i, 0, 0)),
        ],
        out_shape=[
            jax.ShapeDtypeStruct((NCORES, C, C), jnp.float32),
            jax.ShapeDtypeStruct((NCORES, C, 1), jnp.float32),
        ],
        compiler_params=pltpu.CompilerParams(
            dimension_semantics=("parallel", "arbitrary"),
        ),
        name="iternorm_stats",
    )(X)

    out = pl.pallas_call(
        functools.partial(_apply_kernel, m_total),
        grid=grid,
        in_specs=[
            pl.BlockSpec((NCORES, C, C), lambda i, j: (0, 0, 0)),
            pl.BlockSpec((NCORES, C, 1), lambda i, j: (0, 0, 0)),
            x_spec,
        ],
        out_specs=x_spec,
        out_shape=jax.ShapeDtypeStruct((B, C, L), jnp.float32),
        scratch_shapes=[
            pltpu.VMEM((C, C), jnp.float32),
            pltpu.VMEM((C, 1), jnp.float32),
        ],
        compiler_params=pltpu.CompilerParams(
            dimension_semantics=("parallel", "arbitrary"),
            vmem_limit_bytes=56 * 1024 * 1024,
        ),
        name="iternorm_apply",
    )(gram_p, sum_p, X)

    return out
